# baseline (speedup 1.0000x reference)
import functools

import jax
import jax.numpy as jnp
from jax import lax
from jax.experimental import pallas as pl
from jax.experimental.pallas import tpu as pltpu
from jax.experimental.pallas import tpu_sc as plsc

NUM_FIELDS = 26
VOCAB = 100000
EMB_DIM = 32
BATCH = 16384
NUM_DIM = 13
EPS = 1e-5
OUT_DIM = NUM_FIELDS * EMB_DIM + NUM_DIM

NC = 2
NS = 16
NW = NC * NS

NGROUP = (NUM_FIELDS + 3) // 4
GROW = 4 * EMB_DIM

CHUNK = BATCH // NW
GATHER_LEN = 128
GATHERS = CHUNK // GATHER_LEN

VBLK = 2048
VPAD = 100352
BBLK = 512


def _gfields(g):
    return min(4, NUM_FIELDS - 4 * g)


def _regroup_body(tabT_ref, out_ref):
    x = tabT_ref[...]
    out_ref[...] = jnp.swapaxes(x.reshape(GROW, VBLK), 0, 1)


def _regroup(tablesT, g):
    return pl.pallas_call(
        _regroup_body,
        grid=(VPAD // VBLK,),
        in_specs=[
            pl.BlockSpec((4, EMB_DIM, VBLK), lambda j, g=g: (g, 0, j)),
        ],
        out_specs=pl.BlockSpec((VBLK, GROW), lambda j: (j, 0)),
        out_shape=jax.ShapeDtypeStruct((VPAD, GROW), jnp.float32),
    )(tablesT)


def _prep_body(catT_ref, numT_ref, gamma_ref, beta_ref, fidxT_ref,
               numoutT_ref):
    flat = jnp.clip(catT_ref[...], 0, VOCAB - 1).astype(jnp.float32)
    bb = jax.lax.broadcasted_iota(jnp.int32, (BBLK, BBLK), 0)
    pp = jax.lax.broadcasted_iota(jnp.int32, (BBLK, BBLK), 1)
    perm = (bb == (pp % 4) * (BBLK // 4) + pp // 4).astype(jnp.float32)
    v = jnp.dot(flat.reshape(NUM_FIELDS * (BATCH // BBLK), BBLK), perm,
                preferred_element_type=jnp.float32,
                precision=jax.lax.Precision.HIGHEST)
    fidxT_ref[...] = v.reshape(NUM_FIELDS, BATCH).astype(jnp.int32)
    x = numT_ref[...]
    mean = jnp.mean(x, axis=1, keepdims=True)
    var = jnp.mean((x - mean) * (x - mean), axis=1, keepdims=True)
    numoutT_ref[...] = (x - mean) * jax.lax.rsqrt(var + EPS) * gamma_ref[...] \
        + beta_ref[...]


def _prep(catT, numT, bn_gamma, bn_beta):
    return pl.pallas_call(
        _prep_body,
        out_shape=(
            jax.ShapeDtypeStruct((NUM_FIELDS, BATCH), jnp.int32),
            jax.ShapeDtypeStruct((NUM_DIM, BATCH), jnp.float32),
        ),
    )(catT, numT, bn_gamma.reshape(NUM_DIM, 1), bn_beta.reshape(NUM_DIM, 1))


def _gather_body(g, gf, fidx_hbm, table_hbm, out_hbm, idx_v, rows_v, gsem):
    wid = lax.axis_index("s") * NC + lax.axis_index("c")
    for k in range(gf):
        base = k * BATCH + wid * CHUNK
        pltpu.sync_copy(
            fidx_hbm.at[pl.ds((4 * g + k) * BATCH + wid * CHUNK, CHUNK)],
            idx_v)
        handles = []
        for j in range(GATHERS):
            handles.append(pltpu.async_copy(
                table_hbm.at[idx_v.at[pl.ds(j * GATHER_LEN, GATHER_LEN)]],
                rows_v.at[pl.ds(j * GATHER_LEN, GATHER_LEN), :],
                gsem))
        for h in handles:
            h.wait()
        pltpu.sync_copy(rows_v.at[:, pl.ds(k * EMB_DIM, EMB_DIM)],
                        out_hbm.at[pl.ds(base, CHUNK), :])


def _gather(fidx, gtable, g):
    gf = _gfields(g)
    k = functools.partial(
        pl.kernel,
        mesh=plsc.VectorSubcoreMesh(core_axis_name="c", subcore_axis_name="s"),
        out_type=jax.ShapeDtypeStruct((gf * BATCH, EMB_DIM), jnp.float32),
        compiler_params=pltpu.CompilerParams(use_tc_tiling_on_sc=False),
        scratch_types=[
            pltpu.VMEM((CHUNK,), jnp.int32),
            pltpu.VMEM((CHUNK, GROW), jnp.float32),
            pltpu.SemaphoreType.DMA,
        ],
    )(functools.partial(_gather_body, g, gf))
    return k(fidx, gtable)


def _assemble_body(gf, cat_ref, buf_ref, out_ref):
    del buf_ref
    x = cat_ref[...]
    rows = []
    for f in range(gf):
        t = jnp.swapaxes(x[f], 0, 1)
        t = t.reshape(4, EMB_DIM, BBLK // 4)
        rows.append(jnp.swapaxes(t, 0, 1).reshape(EMB_DIM, BBLK))
    out_ref[...] = jnp.concatenate(rows, axis=0)


def _assemble(catG, buf, g):
    gf = _gfields(g)
    band = gf * EMB_DIM
    return pl.pallas_call(
        functools.partial(_assemble_body, gf),
        grid=(BATCH // BBLK,),
        in_specs=[
            pl.BlockSpec((gf, BBLK // 4, GROW), lambda i: (0, i, 0)),
            pl.BlockSpec(memory_space=pl.ANY),
        ],
        out_specs=pl.BlockSpec(
            (band, BBLK), lambda i, g=g: (g * 128 // band, i)),
        out_shape=jax.ShapeDtypeStruct((OUT_DIM, BATCH), jnp.float32),
        input_output_aliases={1: 0},
    )(catG, buf)


def _bn_write_body(num_ref, out_ref):
    out_ref[pl.ds(NUM_FIELDS * EMB_DIM, NUM_DIM), :] = num_ref[...]


def _bn_write(numoutT):
    return pl.pallas_call(
        _bn_write_body,
        grid=(BATCH // BBLK,),
        in_specs=[pl.BlockSpec((NUM_DIM, BBLK), lambda i: (0, i))],
        out_specs=pl.BlockSpec((OUT_DIM, BBLK), lambda i: (0, i)),
        out_shape=jax.ShapeDtypeStruct((OUT_DIM, BATCH), jnp.float32),
    )(numoutT)


def kernel(categorical_inputs, numerical_inputs, tables, bn_gamma, bn_beta):
    catT = categorical_inputs.T
    numT = numerical_inputs.T
    tablesT = tables.transpose(0, 2, 1)
    fidxT, numoutT = _prep(catT, numT, bn_gamma, bn_beta)
    fidx = fidxT.reshape(NUM_FIELDS * BATCH)
    buf = _bn_write(numoutT)
    for g in range(NGROUP):
        gtable = _regroup(tablesT, g)
        catG = _gather(fidx, gtable, g)
        catP = catG.reshape(_gfields(g) * BATCH * EMB_DIM).reshape(
            _gfields(g), BATCH // 4, GROW)
        buf = _assemble(catP, buf, g)
    return buf.T


# speedup vs baseline: 1.0238x; 1.0238x over previous
import functools

import jax
import jax.numpy as jnp
from jax import lax
from jax.experimental import pallas as pl
from jax.experimental.pallas import tpu as pltpu
from jax.experimental.pallas import tpu_sc as plsc

NUM_FIELDS = 26
VOCAB = 100000
EMB_DIM = 32
BATCH = 16384
NUM_DIM = 13
EPS = 1e-5
OUT_DIM = NUM_FIELDS * EMB_DIM + NUM_DIM

NC = 2
NS = 16
NW = NC * NS

GROW = 4 * EMB_DIM
FA = 16
FB = NUM_FIELDS - FA

CHUNK = BATCH // NW
GATHER_LEN = 128
GATHERS = CHUNK // GATHER_LEN

VBLK = 2048
VPAD = 100352
BBLK = 512


def _regroup_body(tabT_ref, out_ref):
    x = tabT_ref[...]
    out_ref[...] = jnp.swapaxes(x.reshape(GROW, VBLK), 0, 1)


def _regroup(tablesT, g0, ng):
    return pl.pallas_call(
        _regroup_body,
        grid=(ng, VPAD // VBLK),
        in_specs=[
            pl.BlockSpec((4, EMB_DIM, VBLK), lambda g, j, g0=g0: (g0 + g, 0, j)),
        ],
        out_specs=pl.BlockSpec(
            (VBLK, GROW), lambda g, j: (g * (VPAD // VBLK) + j, 0)),
        out_shape=jax.ShapeDtypeStruct((ng * VPAD, GROW), jnp.float32),
    )(tablesT)


def _prep_body(catT_ref, numT_ref, gamma_ref, beta_ref, fidxT_ref,
               numoutT_ref):
    idx = jnp.clip(catT_ref[...], 0, VOCAB - 1)
    goff = (jax.lax.broadcasted_iota(jnp.int32, (NUM_FIELDS, 1), 0) // 4) % 4
    flat = (idx + goff * VPAD).astype(jnp.float32)
    bb = jax.lax.broadcasted_iota(jnp.int32, (BBLK, BBLK), 0)
    pp = jax.lax.broadcasted_iota(jnp.int32, (BBLK, BBLK), 1)
    perm = (bb == (pp % 4) * (BBLK // 4) + pp // 4).astype(jnp.float32)
    v = jnp.dot(flat.reshape(NUM_FIELDS * (BATCH // BBLK), BBLK), perm,
                preferred_element_type=jnp.float32,
                precision=jax.lax.Precision.HIGHEST)
    fidxT_ref[...] = v.reshape(NUM_FIELDS, BATCH).astype(jnp.int32)
    x = numT_ref[...]
    mean = jnp.mean(x, axis=1, keepdims=True)
    var = jnp.mean((x - mean) * (x - mean), axis=1, keepdims=True)
    numoutT_ref[...] = (x - mean) * jax.lax.rsqrt(var + EPS) * gamma_ref[...] \
        + beta_ref[...]


def _prep(catT, numT, bn_gamma, bn_beta):
    return pl.pallas_call(
        _prep_body,
        out_shape=(
            jax.ShapeDtypeStruct((NUM_FIELDS, BATCH), jnp.int32),
            jax.ShapeDtypeStruct((NUM_DIM, BATCH), jnp.float32),
        ),
    )(catT, numT, bn_gamma.reshape(NUM_DIM, 1), bn_beta.reshape(NUM_DIM, 1))


def _gather_body(fs, nf, fidx_hbm, table_hbm, out_hbm, idx_v, rows_v, gsem):
    wid = lax.axis_index("s") * NC + lax.axis_index("c")
    for k in range(nf):
        base = k * BATCH + wid * CHUNK
        pltpu.sync_copy(
            fidx_hbm.at[pl.ds((fs + k) * BATCH + wid * CHUNK, CHUNK)],
            idx_v)
        handles = []
        for j in range(GATHERS):
            handles.append(pltpu.async_copy(
                table_hbm.at[idx_v.at[pl.ds(j * GATHER_LEN, GATHER_LEN)]],
                rows_v.at[pl.ds(j * GATHER_LEN, GATHER_LEN), :],
                gsem))
        for h in handles:
            h.wait()
        pltpu.sync_copy(rows_v.at[:, pl.ds((k % 4) * EMB_DIM, EMB_DIM)],
                        out_hbm.at[pl.ds(base, CHUNK), :])


def _gather(fidx, gtable, fs, nf):
    k = functools.partial(
        pl.kernel,
        mesh=plsc.VectorSubcoreMesh(core_axis_name="c", subcore_axis_name="s"),
        out_type=jax.ShapeDtypeStruct((nf * BATCH, EMB_DIM), jnp.float32),
        compiler_params=pltpu.CompilerParams(use_tc_tiling_on_sc=False),
        scratch_types=[
            pltpu.VMEM((CHUNK,), jnp.int32),
            pltpu.VMEM((CHUNK, GROW), jnp.float32),
            pltpu.SemaphoreType.DMA,
        ],
    )(functools.partial(_gather_body, fs, nf))
    return k(fidx, gtable)


def _field_rows(x, f):
    t = jnp.swapaxes(x[f], 0, 1)
    t = t.reshape(4, EMB_DIM, BBLK // 4)
    return jnp.swapaxes(t, 0, 1).reshape(EMB_DIM, BBLK)


def _assemble_a_body(cat_ref, out_ref):
    x = cat_ref[...]
    rows = [_field_rows(x, f) for f in range(FA)]
    out_ref[...] = jnp.concatenate(rows, axis=0)


def _assemble_a(catA):
    return pl.pallas_call(
        _assemble_a_body,
        grid=(BATCH // BBLK,),
        in_specs=[pl.BlockSpec((FA, BBLK // 4, GROW), lambda i: (0, i, 0))],
        out_specs=pl.BlockSpec((FA * EMB_DIM, BBLK), lambda i: (0, i)),
        out_shape=jax.ShapeDtypeStruct((OUT_DIM, BATCH), jnp.float32),
    )(catA)


def _assemble_b_body(cat_ref, num_ref, buf_ref, out_ref):
    del buf_ref
    r = pl.program_id(0)

    @pl.when(r < FB // 2)
    def _fields():
        x = cat_ref[...]
        out_ref[...] = jnp.concatenate([_field_rows(x, 0), _field_rows(x, 1)],
                                       axis=0)

    @pl.when(r == FB // 2)
    def _bn():
        out_ref[pl.ds(0, NUM_DIM), :] = num_ref[...]


def _assemble_b(catB, numoutT, buf):
    nb = FB // 2
    return pl.pallas_call(
        _assemble_b_body,
        grid=(nb + 1, BATCH // BBLK),
        in_specs=[
            pl.BlockSpec((2, BBLK // 4, GROW),
                         lambda r, i: (jnp.minimum(r, nb - 1), i, 0)),
            pl.BlockSpec((NUM_DIM, BBLK), lambda r, i: (0, i)),
            pl.BlockSpec(memory_space=pl.ANY),
        ],
        out_specs=pl.BlockSpec(
            (2 * EMB_DIM, BBLK),
            lambda r, i: (FA * EMB_DIM // (2 * EMB_DIM) + r, i)),
        out_shape=jax.ShapeDtypeStruct((OUT_DIM, BATCH), jnp.float32),
        input_output_aliases={2: 0},
    )(catB, numoutT, buf)


def kernel(categorical_inputs, numerical_inputs, tables, bn_gamma, bn_beta):
    catT = categorical_inputs.T
    numT = numerical_inputs.T
    tablesT = tables.transpose(0, 2, 1)
    fidxT, numoutT = _prep(catT, numT, bn_gamma, bn_beta)
    fidx = fidxT.reshape(NUM_FIELDS * BATCH)
    gtableA = _regroup(tablesT, 0, 4)
    catA = _gather(fidx, gtableA, 0, FA)
    gtableB = _regroup(tablesT, 4, 3)
    catB = _gather(fidx, gtableB, FA, FB)
    catPA = catA.reshape(FA * BATCH * EMB_DIM).reshape(
        FA, BATCH // 4, GROW)
    catPB = catB.reshape(FB * BATCH * EMB_DIM).reshape(
        FB, BATCH // 4, GROW)
    buf = _assemble_a(catPA)
    buf = _assemble_b(catPB, numoutT, buf)
    return buf.T


# speedup vs baseline: 1.0581x; 1.0335x over previous
import functools

import jax
import jax.numpy as jnp
from jax import lax
from jax.experimental import pallas as pl
from jax.experimental.pallas import tpu as pltpu
from jax.experimental.pallas import tpu_sc as plsc

NUM_FIELDS = 26
VOCAB = 100000
EMB_DIM = 32
BATCH = 16384
NUM_DIM = 13
EPS = 1e-5
OUT_DIM = NUM_FIELDS * EMB_DIM + NUM_DIM

NC = 2
NS = 16
NW = NC * NS

GROW = 4 * EMB_DIM
FA = 16
FB = NUM_FIELDS - FA

CHUNK = BATCH // NW
GATHER_LEN = 128
GATHERS = CHUNK // GATHER_LEN

VBLK = 2048
VPAD = 100352
BBLK = 512


def _regroup_body(tabT_ref, out_ref):
    x = tabT_ref[...]
    out_ref[...] = jnp.swapaxes(x.reshape(GROW, VBLK), 0, 1)


def _regroup(tablesT, g0, ng):
    return pl.pallas_call(
        _regroup_body,
        grid=(ng, VPAD // VBLK),
        in_specs=[
            pl.BlockSpec((4, EMB_DIM, VBLK), lambda g, j, g0=g0: (g0 + g, 0, j)),
        ],
        out_specs=pl.BlockSpec(
            (VBLK, GROW), lambda g, j: (g * (VPAD // VBLK) + j, 0)),
        out_shape=jax.ShapeDtypeStruct((ng * VPAD, GROW), jnp.float32),
    )(tablesT)


def _prep_body(catT_ref, numT_ref, gamma_ref, beta_ref, fidxT_ref,
               numoutT_ref):
    idx = jnp.clip(catT_ref[...], 0, VOCAB - 1)
    goff = (jax.lax.broadcasted_iota(jnp.int32, (NUM_FIELDS, 1), 0) // 4) % 4
    flat = (idx + goff * VPAD).astype(jnp.float32)
    bb = jax.lax.broadcasted_iota(jnp.int32, (BBLK, BBLK), 0)
    pp = jax.lax.broadcasted_iota(jnp.int32, (BBLK, BBLK), 1)
    perm = (bb == (pp % 4) * (BBLK // 4) + pp // 4).astype(jnp.float32)
    v = jnp.dot(flat.reshape(NUM_FIELDS * (BATCH // BBLK), BBLK), perm,
                preferred_element_type=jnp.float32,
                precision=jax.lax.Precision.HIGHEST)
    fidxT_ref[...] = v.reshape(NUM_FIELDS, BATCH).astype(jnp.int32)
    x = numT_ref[...]
    mean = jnp.mean(x, axis=1, keepdims=True)
    var = jnp.mean((x - mean) * (x - mean), axis=1, keepdims=True)
    numoutT_ref[...] = (x - mean) * jax.lax.rsqrt(var + EPS) * gamma_ref[...] \
        + beta_ref[...]


def _prep(catT, numT, bn_gamma, bn_beta):
    return pl.pallas_call(
        _prep_body,
        out_shape=(
            jax.ShapeDtypeStruct((NUM_FIELDS, BATCH), jnp.int32),
            jax.ShapeDtypeStruct((NUM_DIM, BATCH), jnp.float32),
        ),
    )(catT, numT, bn_gamma.reshape(NUM_DIM, 1), bn_beta.reshape(NUM_DIM, 1))


def _gather_body(fs, nf, fidx_hbm, table_hbm, out_hbm, idx_v, rows_v, gsem):
    wid = lax.axis_index("s") * NC + lax.axis_index("c")
    for k in range(nf):
        base = k * BATCH + wid * CHUNK
        pltpu.sync_copy(
            fidx_hbm.at[pl.ds((fs + k) * BATCH + wid * CHUNK, CHUNK)],
            idx_v)
        handles = []
        for j in range(GATHERS):
            handles.append(pltpu.async_copy(
                table_hbm.at[idx_v.at[pl.ds(j * GATHER_LEN, GATHER_LEN)]],
                rows_v.at[pl.ds(j * GATHER_LEN, GATHER_LEN), :],
                gsem))
        for h in handles:
            h.wait()
        pltpu.sync_copy(rows_v.at[:, pl.ds((k % 4) * EMB_DIM, EMB_DIM)],
                        out_hbm.at[pl.ds(base, CHUNK), :])


def _gather(fidx, gtable, fs, nf):
    k = functools.partial(
        pl.kernel,
        mesh=plsc.VectorSubcoreMesh(core_axis_name="c", subcore_axis_name="s"),
        out_type=jax.ShapeDtypeStruct((nf * BATCH, EMB_DIM), jnp.float32),
        compiler_params=pltpu.CompilerParams(use_tc_tiling_on_sc=False),
        scratch_types=[
            pltpu.VMEM((CHUNK,), jnp.int32),
            pltpu.VMEM((CHUNK, GROW), jnp.float32),
            pltpu.SemaphoreType.DMA,
        ],
    )(functools.partial(_gather_body, fs, nf))
    return k(fidx, gtable)


def _field_rows(x, f):
    t = jnp.swapaxes(x[f], 0, 1)
    t = t.reshape(4, EMB_DIM, BBLK // 4)
    return jnp.swapaxes(t, 0, 1).reshape(EMB_DIM, BBLK)


def _assemble_a_body(cat_ref, out_ref):
    x = cat_ref[...]
    rows = [_field_rows(x, f) for f in range(FA)]
    out_ref[...] = jnp.concatenate(rows, axis=0)


def _assemble_a(catA):
    return pl.pallas_call(
        _assemble_a_body,
        grid=(BATCH // BBLK,),
        in_specs=[pl.BlockSpec((FA, BBLK // 4, GROW), lambda i: (0, i, 0))],
        out_specs=pl.BlockSpec((FA * EMB_DIM, BBLK), lambda i: (0, i)),
        out_shape=jax.ShapeDtypeStruct((OUT_DIM, BATCH), jnp.float32),
    )(catA)


def _assemble_b_body(cat_ref, buf_ref, out_ref):
    del buf_ref
    x = cat_ref[...]
    out_ref[...] = jnp.concatenate([_field_rows(x, 0), _field_rows(x, 1)],
                                   axis=0)


def _assemble_b(catB, buf):
    return pl.pallas_call(
        _assemble_b_body,
        grid=(FB // 2, BATCH // BBLK),
        in_specs=[
            pl.BlockSpec((2, BBLK // 4, GROW), lambda r, i: (r, i, 0)),
            pl.BlockSpec(memory_space=pl.ANY),
        ],
        out_specs=pl.BlockSpec(
            (2 * EMB_DIM, BBLK),
            lambda r, i: (FA * EMB_DIM // (2 * EMB_DIM) + r, i)),
        out_shape=jax.ShapeDtypeStruct((OUT_DIM, BATCH), jnp.float32),
        input_output_aliases={1: 0},
    )(catB, buf)


def _bn_write_body(num_ref, buf_ref, out_ref):
    del buf_ref
    out_ref[pl.ds(0, NUM_DIM), :] = num_ref[...]


def _bn_write(numoutT, buf):
    return pl.pallas_call(
        _bn_write_body,
        grid=(BATCH // BBLK,),
        in_specs=[
            pl.BlockSpec((NUM_DIM, BBLK), lambda i: (0, i)),
            pl.BlockSpec(memory_space=pl.ANY),
        ],
        out_specs=pl.BlockSpec(
            (2 * EMB_DIM, BBLK),
            lambda i: (NUM_FIELDS * EMB_DIM // (2 * EMB_DIM), i)),
        out_shape=jax.ShapeDtypeStruct((OUT_DIM, BATCH), jnp.float32),
        input_output_aliases={1: 0},
    )(numoutT, buf)


def kernel(categorical_inputs, numerical_inputs, tables, bn_gamma, bn_beta):
    catT = categorical_inputs.T
    numT = numerical_inputs.T
    tablesT = tables.transpose(0, 2, 1)
    fidxT, numoutT = _prep(catT, numT, bn_gamma, bn_beta)
    fidx = fidxT.reshape(NUM_FIELDS * BATCH)
    gtableA = _regroup(tablesT, 0, 4)
    catA = _gather(fidx, gtableA, 0, FA)
    gtableB = _regroup(tablesT, 4, 3)
    catB = _gather(fidx, gtableB, FA, FB)
    catPA = catA.reshape(FA * BATCH * EMB_DIM).reshape(
        FA, BATCH // 4, GROW)
    catPB = catB.reshape(FB * BATCH * EMB_DIM).reshape(
        FB, BATCH // 4, GROW)
    buf = _assemble_a(catPA)
    buf = _bn_write(numoutT, buf)
    buf = _assemble_b(catPB, buf)
    return buf.T
